# baseline (speedup 1.0000x reference)
import functools

import jax
import jax.numpy as jnp
from jax import lax
from jax.experimental import pallas as pl
from jax.experimental.pallas import tpu as pltpu
from jax.experimental.pallas import tpu_sc as plsc

HCT = 2048
TEXT_LEN = 4096
VIS_LEN = 1024
AUD_LEN = 1024
TOTAL = TEXT_LEN + VIS_LEN + AUD_LEN
VOCAB = 100000
NEXP = 16

ROW_CHUNK = 512
N_TEXT_CHUNKS = TEXT_LEN // ROW_CHUNK
V_TILE = 1024

NC, NS = 2, 16
NW = NC * NS
SC_ROWS = 13312
ROWS_PER_W = SC_ROWS // NW
SC_CHUNK = 16
N_SC_PAIRS = ROWS_PER_W // (2 * SC_CHUNK)
TC_ROW0 = SC_ROWS // V_TILE
TC_ROWS = VOCAB - SC_ROWS


def _mean_body(text_ref, vis_ref, aud_ref, rw_ref, out_ref, top_ref):
    step = pl.program_id(0)
    part = jnp.sum(text_ref[...], axis=0, keepdims=True)

    @pl.when(step == 0)
    def _init():
        rest = (jnp.sum(vis_ref[...], axis=0, keepdims=True)
                + jnp.sum(aud_ref[...], axis=0, keepdims=True))
        out_ref[...] = part + rest

    @pl.when(step != 0)
    def _acc():
        out_ref[...] += part

    @pl.when(step == N_TEXT_CHUNKS - 1)
    def _fin():
        fused = out_ref[...] * (1.0 / TOTAL)
        out_ref[...] = fused
        logits = lax.dot_general(
            fused, rw_ref[...],
            dimension_numbers=(((1,), (1,)), ((), ())),
            preferred_element_type=jnp.float32,
        )
        a0 = jnp.argmax(logits, axis=1)[0]
        cols = lax.broadcasted_iota(jnp.int32, (1, NEXP), 1)
        masked = jnp.where(cols == a0, -jnp.inf, logits)
        a1 = jnp.argmax(masked, axis=1)[0]
        out2 = lax.broadcasted_iota(jnp.int32, (1, 2), 1)
        top_ref[...] = jnp.where(out2 == 0, a0.astype(jnp.int32), a1.astype(jnp.int32))


def _fused_mean(text, vis, aud, rw):
    return pl.pallas_call(
        _mean_body,
        grid=(N_TEXT_CHUNKS,),
        in_specs=[
            pl.BlockSpec((ROW_CHUNK, HCT), lambda i: (i, 0)),
            pl.BlockSpec((VIS_LEN, HCT), lambda i: (0, 0)),
            pl.BlockSpec((AUD_LEN, HCT), lambda i: (0, 0)),
            pl.BlockSpec((NEXP, HCT), lambda i: (0, 0)),
        ],
        out_specs=[
            pl.BlockSpec((1, HCT), lambda i: (0, 0)),
            pl.BlockSpec((1, 2), lambda i: (0, 0)),
        ],
        out_shape=[
            jax.ShapeDtypeStruct((1, HCT), jnp.float32),
            jax.ShapeDtypeStruct((1, 2), jnp.int32),
        ],
    )(text, vis, aud, rw)


def _sc_body(fused_hbm, w_hbm, b_hbm, out_hbm, fused_v, wbuf, outv, bv, sem0, sem1):
    wid = lax.axis_index("s") * NC + lax.axis_index("c")
    base = wid * ROWS_PER_W
    pltpu.sync_copy(fused_hbm, fused_v)
    pltpu.sync_copy(b_hbm.at[pl.ds(base, ROWS_PER_W)], bv)
    pltpu.async_copy(w_hbm.at[pl.ds(base, SC_CHUNK), :], wbuf.at[0], sem0)

    lanes = lax.iota(jnp.int32, 16)

    def compute(b, out_off):
        zeros = jnp.zeros((16,), jnp.float32)

        def jbody(j, accs):
            j0 = j * 32
            f0 = fused_v[pl.ds(j0, 16)]
            f1 = fused_v[pl.ds(j0 + 16, 16)]
            new = []
            for r in range(SC_CHUNK):
                a = accs[r] + wbuf[b, r, pl.ds(j0, 16)] * f0
                a = a + wbuf[b, r, pl.ds(j0 + 16, 16)] * f1
                new.append(a)
            return tuple(new)

        accs = lax.fori_loop(0, HCT // 32, jbody, (zeros,) * SC_CHUNK)

        gd = lax.GatherDimensionNumbers(
            offset_dims=(), collapsed_slice_dims=(0,), start_index_map=(0,))

        def lane_sum(v):
            for k in (1, 2, 4, 8):
                perm = lax.gather(v, (lanes ^ k)[:, None], gd, slice_sizes=(1,),
                                  mode=lax.GatherScatterMode.PROMISE_IN_BOUNDS)
                v = v + perm
            return v

        res = jnp.zeros((16,), jnp.float32)
        for r in range(SC_CHUNK):
            res = jnp.where(lanes == r, lane_sum(accs[r]), res)
        outv[pl.ds(out_off, SC_CHUNK)] = res

    def pair(i, carry):
        row_odd = base + (2 * i + 1) * SC_CHUNK
        pltpu.async_copy(w_hbm.at[pl.ds(row_odd, SC_CHUNK), :], wbuf.at[1], sem1)
        pltpu.make_async_copy(
            w_hbm.at[pl.ds(row_odd, SC_CHUNK), :], wbuf.at[0], sem0).wait()
        compute(0, (2 * i) * SC_CHUNK)

        @pl.when(i + 1 < N_SC_PAIRS)
        def _prefetch():
            row_next = base + (2 * i + 2) * SC_CHUNK
            pltpu.async_copy(w_hbm.at[pl.ds(row_next, SC_CHUNK), :], wbuf.at[0], sem0)

        pltpu.make_async_copy(
            w_hbm.at[pl.ds(row_odd, SC_CHUNK), :], wbuf.at[1], sem1).wait()
        compute(1, (2 * i + 1) * SC_CHUNK)
        return carry

    lax.fori_loop(0, N_SC_PAIRS, pair, 0)

    for g in range(ROWS_PER_W // 16):
        outv[pl.ds(g * 16, 16)] += bv[pl.ds(g * 16, 16)]
    pltpu.sync_copy(outv, out_hbm.at[pl.ds(base, ROWS_PER_W)])


def _sc_vocab_slice(fused_flat, w, b):
    mesh = plsc.VectorSubcoreMesh(core_axis_name="c", subcore_axis_name="s")
    run = functools.partial(
        pl.kernel,
        mesh=mesh,
        out_type=jax.ShapeDtypeStruct((SC_ROWS,), jnp.float32),
        scratch_types=[
            pltpu.VMEM((HCT,), jnp.float32),
            pltpu.VMEM((2, SC_CHUNK, HCT), jnp.float32),
            pltpu.VMEM((ROWS_PER_W,), jnp.float32),
            pltpu.VMEM((ROWS_PER_W,), jnp.float32),
            pltpu.SemaphoreType.DMA,
            pltpu.SemaphoreType.DMA,
        ],
    )(_sc_body)
    return run(fused_flat, w, b)


def _proj_body(fused_ref, w_ref, b_ref, out_ref):
    acc = lax.dot_general(
        fused_ref[...], w_ref[...],
        dimension_numbers=(((1,), (1,)), ((), ())),
        preferred_element_type=jnp.float32,
    )
    out_ref[...] = acc + b_ref[...]


def _vocab_proj(fused, w, b):
    n_tiles = pl.cdiv(TC_ROWS, V_TILE)
    return pl.pallas_call(
        _proj_body,
        grid=(n_tiles,),
        in_specs=[
            pl.BlockSpec((1, HCT), lambda i: (0, 0)),
            pl.BlockSpec((V_TILE, HCT), lambda i: (i + TC_ROW0, 0)),
            pl.BlockSpec((1, V_TILE), lambda i: (0, i + TC_ROW0)),
        ],
        out_specs=pl.BlockSpec((1, V_TILE), lambda i: (0, i)),
        out_shape=jax.ShapeDtypeStruct((1, TC_ROWS), jnp.float32),
    )(fused, w, b)


def kernel(text_vector, visual_vector, audio_vector, router_weight, output_weight, output_bias):
    fused, topk = _fused_mean(text_vector, visual_vector, audio_vector, router_weight)
    sc_part = _sc_vocab_slice(fused.reshape(HCT), output_weight, output_bias)
    tc_part = _vocab_proj(fused, output_weight, output_bias[None, :])
    logits = jnp.concatenate([sc_part[None, :], tc_part], axis=1)
    return (logits, topk)


# speedup vs baseline: 1.0206x; 1.0206x over previous
import functools

import jax
import jax.numpy as jnp
from jax import lax
from jax.experimental import pallas as pl
from jax.experimental.pallas import tpu as pltpu
from jax.experimental.pallas import tpu_sc as plsc

HCT = 2048
TEXT_LEN = 4096
VIS_LEN = 1024
AUD_LEN = 1024
TOTAL = TEXT_LEN + VIS_LEN + AUD_LEN
VOCAB = 100000
NEXP = 16

ROW_CHUNK = 512
N_TEXT_CHUNKS = TEXT_LEN // ROW_CHUNK
V_TILE = 1024


def _mean_body(text_ref, vis_ref, aud_ref, out_ref):
    step = pl.program_id(0)
    part = jnp.sum(text_ref[...], axis=0, keepdims=True)

    @pl.when(step == 0)
    def _init():
        rest = (jnp.sum(vis_ref[...], axis=0, keepdims=True)
                + jnp.sum(aud_ref[...], axis=0, keepdims=True))
        out_ref[...] = part + rest

    @pl.when(step != 0)
    def _acc():
        out_ref[...] += part

    @pl.when(step == N_TEXT_CHUNKS - 1)
    def _fin():
        out_ref[...] *= (1.0 / TOTAL)


def _fused_mean(text, vis, aud):
    return pl.pallas_call(
        _mean_body,
        grid=(N_TEXT_CHUNKS,),
        in_specs=[
            pl.BlockSpec((ROW_CHUNK, HCT), lambda i: (i, 0)),
            pl.BlockSpec((VIS_LEN, HCT), lambda i: (0, 0)),
            pl.BlockSpec((AUD_LEN, HCT), lambda i: (0, 0)),
        ],
        out_specs=pl.BlockSpec((1, HCT), lambda i: (0, 0)),
        out_shape=jax.ShapeDtypeStruct((1, HCT), jnp.float32),
    )(text, vis, aud)


def _sc_router_body(fused_hbm, rw_hbm, out_hbm, fused_v, rw_v, idx_v, sem0):
    wid = lax.axis_index("s") * 2 + lax.axis_index("c")

    @pl.when(wid == 0)
    def _only_tile0():
        pltpu.sync_copy(fused_hbm, fused_v)
        pltpu.async_copy(rw_hbm, rw_v, sem0).wait()
        lanes = lax.iota(jnp.int32, 16)
        zeros = jnp.zeros((16,), jnp.float32)

        def jbody(j, accs):
            j0 = j * 32
            f0 = fused_v[pl.ds(j0, 16)]
            f1 = fused_v[pl.ds(j0 + 16, 16)]
            new = []
            for r in range(NEXP):
                a = accs[r] + rw_v[r, pl.ds(j0, 16)] * f0
                a = a + rw_v[r, pl.ds(j0 + 16, 16)] * f1
                new.append(a)
            return tuple(new)

        accs = lax.fori_loop(0, HCT // 32, jbody, (zeros,) * NEXP)

        gd = lax.GatherDimensionNumbers(
            offset_dims=(), collapsed_slice_dims=(0,), start_index_map=(0,))

        def lane_sum(v):
            for k in (1, 2, 4, 8):
                perm = lax.gather(v, (lanes ^ k)[:, None], gd, slice_sizes=(1,),
                                  mode=lax.GatherScatterMode.PROMISE_IN_BOUNDS)
                v = v + perm
            return v

        logits = jnp.zeros((16,), jnp.float32)
        for r in range(NEXP):
            logits = jnp.where(lanes == r, lane_sum(accs[r]), logits)

        def butterfly(v, op):
            for k in (1, 2, 4, 8):
                perm = lax.gather(v, (lanes ^ k)[:, None], gd, slice_sizes=(1,),
                                  mode=lax.GatherScatterMode.PROMISE_IN_BOUNDS)
                v = op(v, perm)
            return v

        def argmax16(v):
            mx = butterfly(v, jnp.maximum)
            cand = jnp.where(v == mx, lanes, jnp.full((16,), 16, jnp.int32))
            return butterfly(cand, jnp.minimum)

        i0 = argmax16(logits)
        neg = jnp.full((16,), -jnp.inf, jnp.float32)
        logits2 = jnp.where(lanes == i0, neg, logits)
        i1 = argmax16(logits2)
        idx_v[...] = jnp.where(lanes == 0, i0, i1)
        pltpu.sync_copy(idx_v, out_hbm)


def _sc_router(fused_flat, rw):
    mesh = plsc.VectorSubcoreMesh(core_axis_name="c", subcore_axis_name="s")
    run = functools.partial(
        pl.kernel,
        mesh=mesh,
        out_type=jax.ShapeDtypeStruct((16,), jnp.int32),
        scratch_types=[
            pltpu.VMEM((HCT,), jnp.float32),
            pltpu.VMEM((NEXP, HCT), jnp.float32),
            pltpu.VMEM((16,), jnp.int32),
            pltpu.SemaphoreType.DMA,
        ],
    )(_sc_router_body)
    return run(fused_flat, rw)


def _proj_body(fused_ref, w_ref, b_ref, out_ref):
    acc = lax.dot_general(
        fused_ref[...], w_ref[...],
        dimension_numbers=(((1,), (1,)), ((), ())),
        preferred_element_type=jnp.float32,
    )
    out_ref[...] = acc + b_ref[...]


def _vocab_proj(fused, w, b):
    n_tiles = pl.cdiv(VOCAB, V_TILE)
    return pl.pallas_call(
        _proj_body,
        grid=(n_tiles,),
        in_specs=[
            pl.BlockSpec((1, HCT), lambda i: (0, 0)),
            pl.BlockSpec((V_TILE, HCT), lambda i: (i, 0)),
            pl.BlockSpec((1, V_TILE), lambda i: (0, i)),
        ],
        out_specs=pl.BlockSpec((1, V_TILE), lambda i: (0, i)),
        out_shape=jax.ShapeDtypeStruct((1, VOCAB), jnp.float32),
    )(fused, w, b)


def kernel(text_vector, visual_vector, audio_vector, router_weight, output_weight, output_bias):
    fused = _fused_mean(text_vector, visual_vector, audio_vector)
    top_idx = _sc_router(fused.reshape(HCT), router_weight)
    logits = _vocab_proj(fused, output_weight, output_bias[None, :])
    topk = top_idx[:2].reshape(1, 2)
    return (logits, topk)


# speedup vs baseline: 1.0703x; 1.0487x over previous
import jax
import jax.numpy as jnp
from jax import lax
from jax.experimental import pallas as pl
from jax.experimental.pallas import tpu as pltpu

HCT = 2048
TEXT_LEN = 4096
VIS_LEN = 1024
AUD_LEN = 1024
TOTAL = TEXT_LEN + VIS_LEN + AUD_LEN
VOCAB = 100000
NEXP = 16

ROW_CHUNK = 512
N_TEXT = TEXT_LEN // ROW_CHUNK
N_VIS = VIS_LEN // ROW_CHUNK
N_AUD = AUD_LEN // ROW_CHUNK
N_MEAN = N_TEXT + N_VIS + N_AUD
V_TILE = 1024
N_PROJ = (VOCAB + V_TILE - 1) // V_TILE


def _body(text_ref, vis_ref, aud_ref, rw_ref, w_ref, b_ref,
          out_ref, top_ref, fused_ref):
    i = pl.program_id(0)

    @pl.when(i < N_TEXT)
    def _text_phase():
        part = jnp.sum(text_ref[...], axis=0, keepdims=True)

        @pl.when(i == 0)
        def _():
            fused_ref[...] = part

        @pl.when(i != 0)
        def _():
            fused_ref[...] += part

    @pl.when((i >= N_TEXT) & (i < N_TEXT + N_VIS))
    def _vis_phase():
        fused_ref[...] += jnp.sum(vis_ref[...], axis=0, keepdims=True)

    @pl.when((i >= N_TEXT + N_VIS) & (i < N_MEAN))
    def _aud_phase():
        fused_ref[...] += jnp.sum(aud_ref[...], axis=0, keepdims=True)

    @pl.when(i == N_MEAN - 1)
    def _finalize_mean_and_route():
        fused = fused_ref[...] * (1.0 / TOTAL)
        fused_ref[...] = fused
        logits = lax.dot_general(
            fused, rw_ref[...],
            dimension_numbers=(((1,), (1,)), ((), ())),
            preferred_element_type=jnp.float32,
        )
        a0 = jnp.argmax(logits, axis=1)[0]
        cols = lax.broadcasted_iota(jnp.int32, (1, NEXP), 1)
        masked = jnp.where(cols == a0, -jnp.inf, logits)
        a1 = jnp.argmax(masked, axis=1)[0]
        out2 = lax.broadcasted_iota(jnp.int32, (1, 2), 1)
        top_ref[...] = jnp.where(out2 == 0, a0.astype(jnp.int32), a1.astype(jnp.int32))

    @pl.when(i >= N_MEAN)
    def _proj_phase():
        acc = lax.dot_general(
            fused_ref[...], w_ref[...],
            dimension_numbers=(((1,), (1,)), ((), ())),
            preferred_element_type=jnp.float32,
        )
        out_ref[...] = acc + b_ref[...]


def _clamp(lo, x, hi):
    return jnp.maximum(lo, jnp.minimum(x, hi))


def kernel(text_vector, visual_vector, audio_vector, router_weight, output_weight, output_bias):
    grid = (N_MEAN + N_PROJ,)
    out_logits, out_top = pl.pallas_call(
        _body,
        grid=grid,
        in_specs=[
            pl.BlockSpec((ROW_CHUNK, HCT), lambda i: (_clamp(0, i, N_TEXT - 1), 0)),
            pl.BlockSpec((ROW_CHUNK, HCT), lambda i: (_clamp(0, i - N_TEXT, N_VIS - 1), 0)),
            pl.BlockSpec((ROW_CHUNK, HCT), lambda i: (_clamp(0, i - N_TEXT - N_VIS, N_AUD - 1), 0)),
            pl.BlockSpec((NEXP, HCT), lambda i: (0, 0)),
            pl.BlockSpec((V_TILE, HCT), lambda i: (_clamp(0, i - N_MEAN, N_PROJ - 1), 0)),
            pl.BlockSpec((1, V_TILE), lambda i: (0, _clamp(0, i - N_MEAN, N_PROJ - 1))),
        ],
        out_specs=[
            pl.BlockSpec((1, V_TILE), lambda i: (0, _clamp(0, i - N_MEAN, N_PROJ - 1))),
            pl.BlockSpec((1, 2), lambda i: (0, 0)),
        ],
        out_shape=[
            jax.ShapeDtypeStruct((1, VOCAB), jnp.float32),
            jax.ShapeDtypeStruct((1, 2), jnp.int32),
        ],
        scratch_shapes=[pltpu.VMEM((1, HCT), jnp.float32)],
    )(text_vector, visual_vector, audio_vector, router_weight,
      output_weight, output_bias[None, :])
    return (out_logits, out_top)


# speedup vs baseline: 1.0823x; 1.0112x over previous
import jax
import jax.numpy as jnp
from jax import lax
from jax.experimental import pallas as pl

HCT = 2048
TEXT_LEN = 4096
VIS_LEN = 1024
AUD_LEN = 1024
TOTAL = TEXT_LEN + VIS_LEN + AUD_LEN
VOCAB = 100000
NEXP = 16

ROW_CHUNK = 512
N_TEXT_CHUNKS = TEXT_LEN // ROW_CHUNK
V_TILE = 1024


def _mean_body(text_ref, vis_ref, aud_ref, rw_ref, out_ref, top_ref):
    step = pl.program_id(0)
    part = jnp.sum(text_ref[...], axis=0, keepdims=True)

    @pl.when(step == 0)
    def _init():
        rest = (jnp.sum(vis_ref[...], axis=0, keepdims=True)
                + jnp.sum(aud_ref[...], axis=0, keepdims=True))
        out_ref[...] = part + rest

    @pl.when(step != 0)
    def _acc():
        out_ref[...] += part

    @pl.when(step == N_TEXT_CHUNKS - 1)
    def _fin():
        fused = out_ref[...] * (1.0 / TOTAL)
        out_ref[...] = fused
        logits = lax.dot_general(
            fused, rw_ref[...],
            dimension_numbers=(((1,), (1,)), ((), ())),
            preferred_element_type=jnp.float32,
        )
        a0 = jnp.argmax(logits, axis=1)[0]
        cols = lax.broadcasted_iota(jnp.int32, (1, NEXP), 1)
        masked = jnp.where(cols == a0, -jnp.inf, logits)
        a1 = jnp.argmax(masked, axis=1)[0]
        out2 = lax.broadcasted_iota(jnp.int32, (1, 2), 1)
        top_ref[...] = jnp.where(out2 == 0, a0.astype(jnp.int32), a1.astype(jnp.int32))


def _fused_mean(text, vis, aud, rw):
    return pl.pallas_call(
        _mean_body,
        grid=(N_TEXT_CHUNKS,),
        in_specs=[
            pl.BlockSpec((ROW_CHUNK, HCT), lambda i: (i, 0)),
            pl.BlockSpec((VIS_LEN, HCT), lambda i: (0, 0)),
            pl.BlockSpec((AUD_LEN, HCT), lambda i: (0, 0)),
            pl.BlockSpec((NEXP, HCT), lambda i: (0, 0)),
        ],
        out_specs=[
            pl.BlockSpec((1, HCT), lambda i: (0, 0)),
            pl.BlockSpec((1, 2), lambda i: (0, 0)),
        ],
        out_shape=[
            jax.ShapeDtypeStruct((1, HCT), jnp.float32),
            jax.ShapeDtypeStruct((1, 2), jnp.int32),
        ],
    )(text, vis, aud, rw)


def _proj_body(fused_ref, w_ref, b_ref, out_ref):
    acc = lax.dot_general(
        fused_ref[...], w_ref[...],
        dimension_numbers=(((1,), (1,)), ((), ())),
        preferred_element_type=jnp.float32,
    )
    out_ref[...] = acc + b_ref[...]


def _vocab_proj(fused, w, b):
    n_tiles = pl.cdiv(VOCAB, V_TILE)
    return pl.pallas_call(
        _proj_body,
        grid=(n_tiles,),
        in_specs=[
            pl.BlockSpec((1, HCT), lambda i: (0, 0)),
            pl.BlockSpec((V_TILE, HCT), lambda i: (i, 0)),
            pl.BlockSpec((1, V_TILE), lambda i: (0, i)),
        ],
        out_specs=pl.BlockSpec((1, V_TILE), lambda i: (0, i)),
        out_shape=jax.ShapeDtypeStruct((1, VOCAB), jnp.float32),
    )(fused, w, b)


def kernel(text_vector, visual_vector, audio_vector, router_weight, output_weight, output_bias):
    fused, topk = _fused_mean(text_vector, visual_vector, audio_vector, router_weight)
    logits = _vocab_proj(fused, output_weight, output_bias[None, :])
    return (logits, topk)
